# baseline (speedup 1.0000x reference)
import functools

import jax
import jax.numpy as jnp
import numpy as np
from jax.experimental import pallas as pl
from jax.experimental.pallas import tpu as pltpu

_H = 16
_DN, _DR, _DV = 128, 64, 128
_DQK = _DN + _DR
_IDX_H, _IDX_D = 32, 128
_TOPK = 1024
_NEG = -1e9


def _dotT(a, b):
    return jax.lax.dot_general(a, b, (((1,), (1,)), ((), ())),
                               preferred_element_type=jnp.float32)


def _rope_int(x, c2, s2):
    n = x.shape[1]
    even = jax.lax.broadcasted_iota(jnp.int32, x.shape, 1) % 2 == 0
    up = pltpu.roll(x, n - 1, 1)
    dn = pltpu.roll(x, 1, 1)
    rot = jnp.where(even, -up, dn)
    return x * c2 + rot * s2


def _stage1_body(x_ref, wqa_ref, qnw_ref, wkva_ref, kvnw_ref, wki_ref,
                 lnw_ref, lnb_ref, cos_ref, sin_ref,
                 cq_ref, ckv_ref, krope_ref, ki_ref):
    xb = x_ref[...]
    c2 = cos_ref[...]
    s2 = sin_ref[...]

    t = _dotT(xb, wqa_ref[...])
    cq_ref[...] = t * jax.lax.rsqrt(
        jnp.mean(t * t, axis=1, keepdims=True) + 1e-6) * qnw_ref[...]

    kv = _dotT(xb, wkva_ref[...])
    ck = kv[:, :512]
    ckv_ref[...] = ck * jax.lax.rsqrt(
        jnp.mean(ck * ck, axis=1, keepdims=True) + 1e-6) * kvnw_ref[...]
    krope_ref[...] = _rope_int(kv[:, 512:], c2, s2)

    kt = _dotT(xb, wki_ref[...])
    m = jnp.mean(kt, axis=1, keepdims=True)
    v = jnp.mean((kt - m) * (kt - m), axis=1, keepdims=True)
    kn = (kt - m) * jax.lax.rsqrt(v + 1e-5) * lnw_ref[...] + lnb_ref[...]
    ki_ref[...] = jnp.concatenate(
        [_rope_int(kn[:, :_DR], c2, s2), kn[:, _DR:]], axis=1)


def _stage3_body(cq_ref, x_ref, wqi_ref, ww_ref, ki_ref, cos_ref, sin_ref,
                 mask_ref, *, bm, seq, k):
    i = pl.program_id(0)

    @pl.when(i * bm + bm <= k)
    def _all_keep():
        mask_ref[...] = jnp.ones((bm, seq), jnp.int8)

    @pl.when(i * bm + bm > k)
    def _select():
        c2 = cos_ref[...]
        s2 = sin_ref[...]
        wts = _dotT(x_ref[...], ww_ref[...]) * (_IDX_H ** -0.5)
        qi = _dotT(cq_ref[...], wqi_ref[...])
        kif = ki_ref[...]
        sc = jnp.zeros((bm, seq), jnp.float32)
        for h in range(_IDX_H):
            qih = qi[:, h * _IDX_D:(h + 1) * _IDX_D]
            qih = jnp.concatenate(
                [_rope_int(qih[:, :_DR], c2, s2), qih[:, _DR:]], axis=1)
            sc = sc + (_dotT(qih, kif) * (_IDX_D ** -0.5)) * wts[:, h:h + 1]
        row = i * bm + jax.lax.broadcasted_iota(jnp.int32, (bm, seq), 0)
        col = jax.lax.broadcasted_iota(jnp.int32, (bm, seq), 1)
        sc = sc + jnp.where(col <= row, 0.0, _NEG)

        u = jax.lax.bitcast_convert_type(sc, jnp.uint32)
        sign = u >= jnp.uint32(0x80000000)
        su = jnp.where(sign, ~u, u | jnp.uint32(0x80000000))

        T = jnp.zeros((bm, 1), jnp.uint32)
        for b in range(31, -1, -1):
            Tt = T | jnp.uint32(2 ** b)
            cnt = jnp.sum((su >= Tt).astype(jnp.int32), axis=1, keepdims=True)
            T = jnp.where(cnt >= k, Tt, T)
        mask_ref[...] = (su >= T).astype(jnp.int8)


def _stage4a_body(cq_ref, wqb_ref, cos_ref, sin_ref, qn_ref, qr_ref):
    cqb = cq_ref[...]
    c2 = cos_ref[...]
    s2 = sin_ref[...]
    for h in range(_H):
        q = _dotT(cqb, wqb_ref[h * _DQK:(h + 1) * _DQK, :])
        qn_ref[h] = q[:, :_DN]
        qr_ref[h] = _rope_int(q[:, _DN:], c2, s2)


def _stage4b_body(ckv_ref, wkvb_ref, kn_ref, v_ref):
    ckvb = ckv_ref[...]
    for h in range(_H):
        kvh = _dotT(ckvb, wkvb_ref[h * (_DN + _DV):(h + 1) * (_DN + _DV), :])
        kn_ref[h] = kvh[:, :_DN]
        v_ref[h] = kvh[:, _DN:]


def _stage5_body(qn_ref, qr_ref, kn_ref, v_ref, krope_ref, mask_ref,
                 out_ref, *, bm, seq):
    i = pl.program_id(1)
    qnb = qn_ref[0]
    qrb = qr_ref[0]
    row = i * bm + jax.lax.broadcasted_iota(jnp.int32, (bm, bm), 0)
    colb = jax.lax.broadcasted_iota(jnp.int32, (bm, bm), 1)

    def body(j, carry):
        m, l, acc = carry
        off = pl.multiple_of(j * bm, bm)
        knj = kn_ref[0, pl.ds(off, bm), :]
        vj = v_ref[0, pl.ds(off, bm), :]
        krj = krope_ref[pl.ds(off, bm), :]
        mkj = mask_ref[:, pl.ds(off, bm)]
        att = (_dotT(qnb, knj) + _dotT(qrb, krj)) * (_DQK ** -0.5)
        keep = jnp.logical_and(j * bm + colb <= row, mkj != 0)
        att = jnp.where(keep, att, -1e30)
        mn = jnp.maximum(m, jnp.max(att, axis=1, keepdims=True))
        corr = jnp.exp(m - mn)
        p = jnp.where(keep, jnp.exp(att - mn), 0.0)
        l = l * corr + jnp.sum(p, axis=1, keepdims=True)
        acc = acc * corr + jnp.dot(p, vj, preferred_element_type=jnp.float32)
        return mn, l, acc

    m0 = jnp.full((bm, 1), -1e30, jnp.float32)
    l0 = jnp.zeros((bm, 1), jnp.float32)
    a0 = jnp.zeros((bm, _DV), jnp.float32)
    m, l, acc = jax.lax.fori_loop(0, i + 1, body, (m0, l0, a0))
    out_ref[0] = acc / l


def _stage6_body(attn_ref, wo_ref, out_ref):
    acc = _dotT(attn_ref[0], wo_ref[:, :_DV])
    for h in range(1, _H):
        acc = acc + _dotT(attn_ref[h], wo_ref[:, h * _DV:(h + 1) * _DV])
    out_ref[...] = acc


def kernel(x, freqs_cos, freqs_sin, wq_a, q_norm_w, wq_b, wkv_a, kv_norm_w,
           wkv_b, wo, idx_wq, idx_wk, idx_ln_w, idx_ln_b, idx_ww):
    b, seq, dm = x.shape
    x2 = x[0]
    bm = 256
    nblk = seq // bm
    k = min(_TOPK, seq)
    q_lora = wq_a.shape[0]
    kv_lora = kv_norm_w.shape[0]

    c2 = jnp.repeat(freqs_cos, 2, axis=1)
    s2 = jnp.repeat(freqs_sin, 2, axis=1)
    qnw = q_norm_w[None, :]
    kvnw = kv_norm_w[None, :]
    lnw = idx_ln_w[None, :]
    lnb = idx_ln_b[None, :]

    f32 = jnp.float32
    row_spec = lambda w: pl.BlockSpec((bm, w), lambda i: (i, 0))
    full_spec = lambda a: pl.BlockSpec(a.shape, lambda *_: (0,) * a.ndim)

    cq, ckv, krope, ki = pl.pallas_call(
        _stage1_body,
        grid=(nblk,),
        in_specs=[row_spec(dm), full_spec(wq_a), full_spec(qnw),
                  full_spec(wkv_a), full_spec(kvnw), full_spec(idx_wk),
                  full_spec(lnw), full_spec(lnb),
                  row_spec(_DR), row_spec(_DR)],
        out_specs=[row_spec(q_lora), row_spec(kv_lora), row_spec(_DR),
                   row_spec(_IDX_D)],
        out_shape=[jax.ShapeDtypeStruct((seq, q_lora), f32),
                   jax.ShapeDtypeStruct((seq, kv_lora), f32),
                   jax.ShapeDtypeStruct((seq, _DR), f32),
                   jax.ShapeDtypeStruct((seq, _IDX_D), f32)],
    )(x2, wq_a, qnw, wkv_a, kvnw, idx_wk, lnw, lnb, c2, s2)

    mask = pl.pallas_call(
        functools.partial(_stage3_body, bm=bm, seq=seq, k=k),
        grid=(nblk,),
        in_specs=[row_spec(q_lora), row_spec(dm), full_spec(idx_wq),
                  full_spec(idx_ww), full_spec(ki),
                  row_spec(_DR), row_spec(_DR)],
        out_specs=row_spec(seq),
        out_shape=jax.ShapeDtypeStruct((seq, seq), jnp.int8),
    )(cq, x2, idx_wq, idx_ww, ki, c2, s2)

    head_row = lambda w: pl.BlockSpec((_H, bm, w), lambda i: (0, i, 0))
    qn, qr = pl.pallas_call(
        _stage4a_body,
        grid=(nblk,),
        in_specs=[row_spec(q_lora), full_spec(wq_b),
                  row_spec(_DR), row_spec(_DR)],
        out_specs=[head_row(_DN), head_row(_DR)],
        out_shape=[jax.ShapeDtypeStruct((_H, seq, _DN), f32),
                   jax.ShapeDtypeStruct((_H, seq, _DR), f32)],
    )(cq, wq_b, c2, s2)

    kn, v = pl.pallas_call(
        _stage4b_body,
        grid=(nblk,),
        in_specs=[row_spec(kv_lora), full_spec(wkv_b)],
        out_specs=[head_row(_DN), head_row(_DV)],
        out_shape=[jax.ShapeDtypeStruct((_H, seq, _DN), f32),
                   jax.ShapeDtypeStruct((_H, seq, _DV), f32)],
    )(ckv, wkv_b)

    hblk = lambda w: pl.BlockSpec((1, bm, w), lambda h, i: (h, i, 0))
    hfull = lambda w: pl.BlockSpec((1, seq, w), lambda h, i: (h, 0, 0))
    attn = pl.pallas_call(
        functools.partial(_stage5_body, bm=bm, seq=seq),
        grid=(_H, nblk),
        in_specs=[hblk(_DN), hblk(_DR), hfull(_DN), hfull(_DV),
                  pl.BlockSpec((seq, _DR), lambda h, i: (0, 0)),
                  pl.BlockSpec((bm, seq), lambda h, i: (i, 0))],
        out_specs=hblk(_DV),
        out_shape=jax.ShapeDtypeStruct((_H, seq, _DV), f32),
    )(qn, qr, kn, v, krope, mask)

    out = pl.pallas_call(
        _stage6_body,
        grid=(nblk,),
        in_specs=[head_row(_DV), full_spec(wo)],
        out_specs=row_spec(dm),
        out_shape=jax.ShapeDtypeStruct((seq, dm), f32),
    )(attn, wo)

    return out[None]


# speedup vs baseline: 1.1158x; 1.1158x over previous
import functools

import jax
import jax.numpy as jnp
import numpy as np
from jax.experimental import pallas as pl
from jax.experimental.pallas import tpu as pltpu

_H = 16
_DN, _DR, _DV = 128, 64, 128
_DQK = _DN + _DR
_IDX_H, _IDX_D = 32, 128
_TOPK = 1024
_NEG = -1e9


def _dotT(a, b):
    return jax.lax.dot_general(a, b, (((1,), (1,)), ((), ())),
                               preferred_element_type=jnp.float32)


def _rope_int(x, c2, s2):
    n = x.shape[1]
    even = jax.lax.broadcasted_iota(jnp.int32, x.shape, 1) % 2 == 0
    up = pltpu.roll(x, n - 1, 1)
    dn = pltpu.roll(x, 1, 1)
    rot = jnp.where(even, -up, dn)
    return x * c2 + rot * s2


def _stage1_body(x_ref, wqa_ref, qnw_ref, wkva_ref, kvnw_ref, wki_ref,
                 lnw_ref, lnb_ref, cos_ref, sin_ref,
                 cq_ref, ckv_ref, krope_ref, ki_ref):
    xb = x_ref[...]
    c2 = cos_ref[...]
    s2 = sin_ref[...]

    t = _dotT(xb, wqa_ref[...])
    cq_ref[...] = t * jax.lax.rsqrt(
        jnp.mean(t * t, axis=1, keepdims=True) + 1e-6) * qnw_ref[...]

    kv = _dotT(xb, wkva_ref[...])
    ck = kv[:, :512]
    ckv_ref[...] = ck * jax.lax.rsqrt(
        jnp.mean(ck * ck, axis=1, keepdims=True) + 1e-6) * kvnw_ref[...]
    krope_ref[...] = _rope_int(kv[:, 512:], c2, s2)

    kt = _dotT(xb, wki_ref[...])
    m = jnp.mean(kt, axis=1, keepdims=True)
    v = jnp.mean((kt - m) * (kt - m), axis=1, keepdims=True)
    kn = (kt - m) * jax.lax.rsqrt(v + 1e-5) * lnw_ref[...] + lnb_ref[...]
    ki_ref[...] = jnp.concatenate(
        [_rope_int(kn[:, :_DR], c2, s2), kn[:, _DR:]], axis=1)


def _stage3_body(cq_ref, x_ref, wqi_ref, ww_ref, ki_ref, cos_ref, sin_ref,
                 mask_ref, *, bm, seq, k):
    i = pl.program_id(0)

    @pl.when(i * bm + bm <= k)
    def _all_keep():
        mask_ref[...] = jnp.ones((bm, seq), jnp.int8)

    @pl.when(i * bm + bm > k)
    def _select():
        c2 = cos_ref[...]
        s2 = sin_ref[...]
        wts = _dotT(x_ref[...], ww_ref[...]) * (_IDX_H ** -0.5)
        w2 = wts * (_IDX_D ** -0.5)
        qi = _dotT(cq_ref[...], wqi_ref[...])
        kif = ki_ref[...]
        sc = jnp.zeros((bm, seq), jnp.float32)
        for h in range(_IDX_H):
            qih = qi[:, h * _IDX_D:(h + 1) * _IDX_D]
            qih = jnp.concatenate(
                [_rope_int(qih[:, :_DR], c2, s2), qih[:, _DR:]], axis=1)
            sc = sc + _dotT(qih, kif) * w2[:, h:h + 1]
        row = i * bm + jax.lax.broadcasted_iota(jnp.int32, (bm, seq), 0)
        col = jax.lax.broadcasted_iota(jnp.int32, (bm, seq), 1)
        sc = sc + jnp.where(col <= row, 0.0, _NEG)

        u = jax.lax.bitcast_convert_type(sc, jnp.uint32)
        sign = u >= jnp.uint32(0x80000000)
        su = jnp.where(sign, ~u, u | jnp.uint32(0x80000000))

        T = jnp.zeros((bm, 1), jnp.uint32)
        for b in range(31, -1, -1):
            Tt = T | jnp.uint32(2 ** b)
            cnt = jnp.sum((su >= Tt).astype(jnp.int32), axis=1, keepdims=True)
            T = jnp.where(cnt >= k, Tt, T)
        mask_ref[...] = (su >= T).astype(jnp.int8)


def _stage4a_body(cq_ref, wqb_ref, cos_ref, sin_ref, qn_ref, qr_ref):
    cqb = cq_ref[...]
    c2 = cos_ref[...]
    s2 = sin_ref[...]
    for h in range(_H):
        q = _dotT(cqb, wqb_ref[h * _DQK:(h + 1) * _DQK, :])
        qn_ref[h] = q[:, :_DN]
        qr_ref[h] = _rope_int(q[:, _DN:], c2, s2)


def _stage4b_body(ckv_ref, wkvb_ref, kn_ref, v_ref):
    ckvb = ckv_ref[...]
    for h in range(_H):
        kvh = _dotT(ckvb, wkvb_ref[h * (_DN + _DV):(h + 1) * (_DN + _DV), :])
        kn_ref[h] = kvh[:, :_DN]
        v_ref[h] = kvh[:, _DN:]


def _stage5_body(qn_ref, qr_ref, kn_ref, v_ref, krope_ref, mask_ref,
                 out_ref, *, bm, seq):
    i = pl.program_id(1)
    att = _dotT(qn_ref[0], kn_ref[0]) + _dotT(qr_ref[0], krope_ref[...])
    att = att * (_DQK ** -0.5)

    row = i * bm + jax.lax.broadcasted_iota(jnp.int32, (bm, seq), 0)
    col = jax.lax.broadcasted_iota(jnp.int32, (bm, seq), 1)
    keep = jnp.logical_and(col <= row, mask_ref[...] != 0)

    att = jnp.where(keep, att, _NEG)
    m = jnp.max(att, axis=1, keepdims=True)
    p = jnp.exp(att - m)
    p = p / jnp.sum(p, axis=1, keepdims=True)
    out_ref[...] = jnp.dot(p, v_ref[0], preferred_element_type=jnp.float32)


def _stage6_body(attn_ref, wo_ref, out_ref):
    out_ref[...] = _dotT(attn_ref[...], wo_ref[...])


def kernel(x, freqs_cos, freqs_sin, wq_a, q_norm_w, wq_b, wkv_a, kv_norm_w,
           wkv_b, wo, idx_wq, idx_wk, idx_ln_w, idx_ln_b, idx_ww):
    b, seq, dm = x.shape
    x2 = x[0]
    bm = 256
    nblk = seq // bm
    k = min(_TOPK, seq)
    q_lora = wq_a.shape[0]
    kv_lora = kv_norm_w.shape[0]

    c2 = jnp.repeat(freqs_cos, 2, axis=1)
    s2 = jnp.repeat(freqs_sin, 2, axis=1)
    qnw = q_norm_w[None, :]
    kvnw = kv_norm_w[None, :]
    lnw = idx_ln_w[None, :]
    lnb = idx_ln_b[None, :]

    f32 = jnp.float32
    row_spec = lambda w: pl.BlockSpec((bm, w), lambda i: (i, 0))
    full_spec = lambda a: pl.BlockSpec(a.shape, lambda *_: (0,) * a.ndim)

    cq, ckv, krope, ki = pl.pallas_call(
        _stage1_body,
        grid=(nblk,),
        in_specs=[row_spec(dm), full_spec(wq_a), full_spec(qnw),
                  full_spec(wkv_a), full_spec(kvnw), full_spec(idx_wk),
                  full_spec(lnw), full_spec(lnb),
                  row_spec(_DR), row_spec(_DR)],
        out_specs=[row_spec(q_lora), row_spec(kv_lora), row_spec(_DR),
                   row_spec(_IDX_D)],
        out_shape=[jax.ShapeDtypeStruct((seq, q_lora), f32),
                   jax.ShapeDtypeStruct((seq, kv_lora), f32),
                   jax.ShapeDtypeStruct((seq, _DR), f32),
                   jax.ShapeDtypeStruct((seq, _IDX_D), f32)],
    )(x2, wq_a, qnw, wkv_a, kvnw, idx_wk, lnw, lnb, c2, s2)

    mask = pl.pallas_call(
        functools.partial(_stage3_body, bm=bm, seq=seq, k=k),
        grid=(nblk,),
        in_specs=[row_spec(q_lora), row_spec(dm), full_spec(idx_wq),
                  full_spec(idx_ww), full_spec(ki),
                  row_spec(_DR), row_spec(_DR)],
        out_specs=row_spec(seq),
        out_shape=jax.ShapeDtypeStruct((seq, seq), jnp.int8),
    )(cq, x2, idx_wq, idx_ww, ki, c2, s2)

    head_row = lambda w: pl.BlockSpec((_H, bm, w), lambda i: (0, i, 0))
    qn, qr = pl.pallas_call(
        _stage4a_body,
        grid=(nblk,),
        in_specs=[row_spec(q_lora), full_spec(wq_b),
                  row_spec(_DR), row_spec(_DR)],
        out_specs=[head_row(_DN), head_row(_DR)],
        out_shape=[jax.ShapeDtypeStruct((_H, seq, _DN), f32),
                   jax.ShapeDtypeStruct((_H, seq, _DR), f32)],
    )(cq, wq_b, c2, s2)

    kn, v = pl.pallas_call(
        _stage4b_body,
        grid=(nblk,),
        in_specs=[row_spec(kv_lora), full_spec(wkv_b)],
        out_specs=[head_row(_DN), head_row(_DV)],
        out_shape=[jax.ShapeDtypeStruct((_H, seq, _DN), f32),
                   jax.ShapeDtypeStruct((_H, seq, _DV), f32)],
    )(ckv, wkv_b)

    hblk = lambda w: pl.BlockSpec((1, bm, w), lambda h, i: (h, i, 0))
    hfull = lambda w: pl.BlockSpec((1, seq, w), lambda h, i: (h, 0, 0))
    attn = pl.pallas_call(
        functools.partial(_stage5_body, bm=bm, seq=seq),
        grid=(_H, nblk),
        in_specs=[hblk(_DN), hblk(_DR), hfull(_DN), hfull(_DV),
                  pl.BlockSpec((seq, _DR), lambda h, i: (0, 0)),
                  pl.BlockSpec((bm, seq), lambda h, i: (i, 0))],
        out_specs=pl.BlockSpec((bm, _DV), lambda h, i: (i, h)),
        out_shape=jax.ShapeDtypeStruct((seq, _H * _DV), f32),
    )(qn, qr, kn, v, krope, mask)

    out = pl.pallas_call(
        _stage6_body,
        grid=(nblk,),
        in_specs=[row_spec(_H * _DV), full_spec(wo)],
        out_specs=row_spec(dm),
        out_shape=jax.ShapeDtypeStruct((seq, dm), f32),
    )(attn, wo)

    return out[None]


# speedup vs baseline: 1.1869x; 1.0638x over previous
import functools

import jax
import jax.numpy as jnp
import numpy as np
from jax.experimental import pallas as pl
from jax.experimental.pallas import tpu as pltpu

_H = 16
_DN, _DR, _DV = 128, 64, 128
_DQK = _DN + _DR
_IDX_H, _IDX_D = 32, 128
_TOPK = 1024
_NEG = -1e9


def _dotT(a, b):
    return jax.lax.dot_general(a, b, (((1,), (1,)), ((), ())),
                               preferred_element_type=jnp.float32)


def _rope_int(x, c2, s2):
    n = x.shape[1]
    even = jax.lax.broadcasted_iota(jnp.int32, x.shape, 1) % 2 == 0
    up = pltpu.roll(x, n - 1, 1)
    dn = pltpu.roll(x, 1, 1)
    rot = jnp.where(even, -up, dn)
    return x * c2 + rot * s2


def _stage1_body(x_ref, wqa_ref, qnw_ref, wkva_ref, kvnw_ref, wki_ref,
                 lnw_ref, lnb_ref, cos_ref, sin_ref,
                 cq_ref, ckv_ref, krope_ref, ki_ref):
    xb = x_ref[...]
    c2 = cos_ref[...]
    s2 = sin_ref[...]

    t = _dotT(xb, wqa_ref[...])
    cq_ref[...] = t * jax.lax.rsqrt(
        jnp.mean(t * t, axis=1, keepdims=True) + 1e-6) * qnw_ref[...]

    kv = _dotT(xb, wkva_ref[...])
    ck = kv[:, :512]
    ckv_ref[...] = ck * jax.lax.rsqrt(
        jnp.mean(ck * ck, axis=1, keepdims=True) + 1e-6) * kvnw_ref[...]
    krope_ref[...] = _rope_int(kv[:, 512:], c2, s2)

    kt = _dotT(xb, wki_ref[...])
    m = jnp.mean(kt, axis=1, keepdims=True)
    v = jnp.mean((kt - m) * (kt - m), axis=1, keepdims=True)
    kn = (kt - m) * jax.lax.rsqrt(v + 1e-5) * lnw_ref[...] + lnb_ref[...]
    ki_ref[...] = jnp.concatenate(
        [_rope_int(kn[:, :_DR], c2, s2), kn[:, _DR:]], axis=1)


def _stage3_body(cq_ref, x_ref, wqi_ref, ww_ref, ki_ref, cos_ref, sin_ref,
                 mask_ref, *, bm, seq, k):
    i = pl.program_id(0)

    @pl.when(i * bm + bm <= k)
    def _all_keep():
        row = i * bm + jax.lax.broadcasted_iota(jnp.int32, (bm, seq), 0)
        col = jax.lax.broadcasted_iota(jnp.int32, (bm, seq), 1)
        mask_ref[...] = (col <= row).astype(jnp.int8)

    @pl.when(i * bm + bm > k)
    def _select():
        c2 = cos_ref[...]
        s2 = sin_ref[...]
        wts = _dotT(x_ref[...], ww_ref[...]) * (_IDX_H ** -0.5)
        w2 = wts * (_IDX_D ** -0.5)
        qi = _dotT(cq_ref[...], wqi_ref[...])
        kif = ki_ref[...]
        sc = jnp.zeros((bm, seq), jnp.float32)
        for h in range(_IDX_H):
            qih = qi[:, h * _IDX_D:(h + 1) * _IDX_D]
            qih = jnp.concatenate(
                [_rope_int(qih[:, :_DR], c2, s2), qih[:, _DR:]], axis=1)
            sc = sc + _dotT(qih, kif) * w2[:, h:h + 1]
        row = i * bm + jax.lax.broadcasted_iota(jnp.int32, (bm, seq), 0)
        col = jax.lax.broadcasted_iota(jnp.int32, (bm, seq), 1)
        sc = sc + jnp.where(col <= row, 0.0, _NEG)

        u = jax.lax.bitcast_convert_type(sc, jnp.uint32)
        sign = u >= jnp.uint32(0x80000000)
        su = jnp.where(sign, ~u, u | jnp.uint32(0x80000000))

        T = jnp.zeros((bm, 1), jnp.uint32)
        for b in range(31, -1, -1):
            Tt = T | jnp.uint32(2 ** b)
            cnt = jnp.sum((su >= Tt).astype(jnp.int32), axis=1, keepdims=True)
            T = jnp.where(cnt >= k, Tt, T)
        mask_ref[...] = jnp.logical_and(su >= T, col <= row).astype(jnp.int8)


def _stage4a_body(cq_ref, wqb_ref, cos_ref, sin_ref, qn_ref, qr_ref):
    cqb = cq_ref[...]
    c2 = cos_ref[...]
    s2 = sin_ref[...]
    for h in range(_H):
        q = _dotT(cqb, wqb_ref[h * _DQK:(h + 1) * _DQK, :])
        qn_ref[h] = q[:, :_DN]
        qr_ref[h] = _rope_int(q[:, _DN:], c2, s2)


def _stage4b_body(ckv_ref, wkvb_ref, kn_ref, v_ref):
    ckvb = ckv_ref[...]
    for h in range(_H):
        kvh = _dotT(ckvb, wkvb_ref[h * (_DN + _DV):(h + 1) * (_DN + _DV), :])
        kn_ref[h] = kvh[:, :_DN]
        v_ref[h] = kvh[:, _DN:]


def _stage5_body(qn_ref, qr_ref, kn_ref, v_ref, krope_ref, mask_ref,
                 out_ref, *, bm, seq):
    att = _dotT(qn_ref[0], kn_ref[0]) + _dotT(qr_ref[0], krope_ref[...])
    att = att * (_DQK ** -0.5)

    att = jnp.where(mask_ref[...] != 0, att, _NEG)
    m = jnp.max(att, axis=1, keepdims=True)
    p = jnp.exp(att - m)
    l = jnp.sum(p, axis=1, keepdims=True)
    out_ref[...] = jnp.dot(p, v_ref[0],
                           preferred_element_type=jnp.float32) / l


def _stage6_body(attn_ref, wo_ref, out_ref):
    out_ref[...] = _dotT(attn_ref[...], wo_ref[...])


def kernel(x, freqs_cos, freqs_sin, wq_a, q_norm_w, wq_b, wkv_a, kv_norm_w,
           wkv_b, wo, idx_wq, idx_wk, idx_ln_w, idx_ln_b, idx_ww):
    b, seq, dm = x.shape
    x2 = x[0]
    bm = 256
    nblk = seq // bm
    k = min(_TOPK, seq)
    q_lora = wq_a.shape[0]
    kv_lora = kv_norm_w.shape[0]

    c2 = jnp.repeat(freqs_cos, 2, axis=1)
    s2 = jnp.repeat(freqs_sin, 2, axis=1)
    qnw = q_norm_w[None, :]
    kvnw = kv_norm_w[None, :]
    lnw = idx_ln_w[None, :]
    lnb = idx_ln_b[None, :]

    f32 = jnp.float32
    row_spec = lambda w: pl.BlockSpec((bm, w), lambda i: (i, 0))
    full_spec = lambda a: pl.BlockSpec(a.shape, lambda *_: (0,) * a.ndim)

    cq, ckv, krope, ki = pl.pallas_call(
        _stage1_body,
        grid=(nblk,),
        in_specs=[row_spec(dm), full_spec(wq_a), full_spec(qnw),
                  full_spec(wkv_a), full_spec(kvnw), full_spec(idx_wk),
                  full_spec(lnw), full_spec(lnb),
                  row_spec(_DR), row_spec(_DR)],
        out_specs=[row_spec(q_lora), row_spec(kv_lora), row_spec(_DR),
                   row_spec(_IDX_D)],
        out_shape=[jax.ShapeDtypeStruct((seq, q_lora), f32),
                   jax.ShapeDtypeStruct((seq, kv_lora), f32),
                   jax.ShapeDtypeStruct((seq, _DR), f32),
                   jax.ShapeDtypeStruct((seq, _IDX_D), f32)],
    )(x2, wq_a, qnw, wkv_a, kvnw, idx_wk, lnw, lnb, c2, s2)

    mask = pl.pallas_call(
        functools.partial(_stage3_body, bm=bm, seq=seq, k=k),
        grid=(nblk,),
        in_specs=[row_spec(q_lora), row_spec(dm), full_spec(idx_wq),
                  full_spec(idx_ww), full_spec(ki),
                  row_spec(_DR), row_spec(_DR)],
        out_specs=row_spec(seq),
        out_shape=jax.ShapeDtypeStruct((seq, seq), jnp.int8),
    )(cq, x2, idx_wq, idx_ww, ki, c2, s2)

    head_row = lambda w: pl.BlockSpec((_H, bm, w), lambda i: (0, i, 0))
    qn, qr = pl.pallas_call(
        _stage4a_body,
        grid=(nblk,),
        in_specs=[row_spec(q_lora), full_spec(wq_b),
                  row_spec(_DR), row_spec(_DR)],
        out_specs=[head_row(_DN), head_row(_DR)],
        out_shape=[jax.ShapeDtypeStruct((_H, seq, _DN), f32),
                   jax.ShapeDtypeStruct((_H, seq, _DR), f32)],
    )(cq, wq_b, c2, s2)

    kn, v = pl.pallas_call(
        _stage4b_body,
        grid=(nblk,),
        in_specs=[row_spec(kv_lora), full_spec(wkv_b)],
        out_specs=[head_row(_DN), head_row(_DV)],
        out_shape=[jax.ShapeDtypeStruct((_H, seq, _DN), f32),
                   jax.ShapeDtypeStruct((_H, seq, _DV), f32)],
    )(ckv, wkv_b)

    hblk = lambda w: pl.BlockSpec((1, bm, w), lambda h, i: (h, i, 0))
    hfull = lambda w: pl.BlockSpec((1, seq, w), lambda h, i: (h, 0, 0))
    attn = pl.pallas_call(
        functools.partial(_stage5_body, bm=bm, seq=seq),
        grid=(_H, nblk),
        in_specs=[hblk(_DN), hblk(_DR), hfull(_DN), hfull(_DV),
                  pl.BlockSpec((seq, _DR), lambda h, i: (0, 0)),
                  pl.BlockSpec((bm, seq), lambda h, i: (i, 0))],
        out_specs=pl.BlockSpec((bm, _DV), lambda h, i: (i, h)),
        out_shape=jax.ShapeDtypeStruct((seq, _H * _DV), f32),
    )(qn, qr, kn, v, krope, mask)

    out = pl.pallas_call(
        _stage6_body,
        grid=(nblk,),
        in_specs=[row_spec(_H * _DV), full_spec(wo)],
        out_specs=row_spec(dm),
        out_shape=jax.ShapeDtypeStruct((seq, dm), f32),
    )(attn, wo)

    return out[None]
